# baseline (speedup 1.0000x reference)
import functools

import jax
import jax.numpy as jnp
from jax import lax
from jax.experimental import pallas as pl
from jax.experimental.pallas import tpu as pltpu
from jax.experimental.pallas import tpu_sc as plsc

NUM_EMB = 1000000
DIM = 64
BATCH = 4096
HIST = 50

NC = 2
NS = 16
NW = NC * NS
CHUNK = 128
NCHUNK = HIST
NBUF = 5
NTBUF = 5


def _body(idx_hbm, table_hbm, out_hbm, idx_v, rows_v, tbuf, gsem, osem):
  wid = lax.axis_index("s") * NC + lax.axis_index("c")
  base_b = wid * CHUNK

  pltpu.sync_copy(idx_hbm.at[:, pl.ds(base_b, CHUNK)], idx_v)

  def start_gather(j, b):
    pltpu.async_copy(table_hbm.at[idx_v.at[j]], rows_v.at[b], gsem.at[b])

  def wait_gather(j, b):
    pltpu.make_async_copy(table_hbm.at[idx_v.at[j]], rows_v.at[b],
                          gsem.at[b]).wait()

  def start_writes(j, tb):
    for dhi in range(8):
      pltpu.async_copy(tbuf.at[tb, dhi], out_hbm.at[j, dhi, wid], osem.at[tb])

  def wait_writes(j, tb):
    for dhi in range(8):
      pltpu.make_async_copy(tbuf.at[tb, dhi], out_hbm.at[j, dhi, wid],
                            osem.at[tb]).wait()

  lanes = lax.iota(jnp.int32, 16)

  def transpose_chunk(b, tb):
    @pl.loop(0, 8)
    def _dhi(dhi):
      for dlo in range(8):
        d = dhi * 8 + dlo
        col = jnp.broadcast_to(d, (16,)).astype(jnp.int32)
        for k in range(8):
          row = lanes + (16 * k)
          v = plsc.load_gather(rows_v.at[b], [row, col])
          tbuf[tb, dhi, dlo, pl.ds(16 * k, 16)] = v

  for b in range(NBUF):
    start_gather(b, b)

  @pl.loop(0, NCHUNK, step=NBUF)
  def _outer(j0):
    for b in range(NBUF):
      j = j0 + b
      bp = (b - 1) % NBUF
      tb = b
      wait_gather(j, b)

      if b == 0:
        @pl.when((j >= 1) & (j + NBUF - 1 < NCHUNK))
        def _():
          start_gather(j + NBUF - 1, bp)
      else:
        @pl.when(j + NBUF - 1 < NCHUNK)
        def _():
          start_gather(j + NBUF - 1, bp)

      @pl.when(j >= NTBUF)
      def _():
        wait_writes(j - NTBUF, tb)

      transpose_chunk(b, tb)
      start_writes(j, tb)

  for b in range(NTBUF):
    wait_writes(NCHUNK - NTBUF + b, b)


@jax.jit
def kernel(indices, table):
  idx_t = indices.T.astype(jnp.int32)
  run = pl.kernel(
      _body,
      out_type=jax.ShapeDtypeStruct((HIST, 8, NW, 8, CHUNK), jnp.float32),
      mesh=plsc.VectorSubcoreMesh(core_axis_name="c", subcore_axis_name="s"),
      compiler_params=pltpu.CompilerParams(use_tc_tiling_on_sc=False,
                                           needs_layout_passes=False),
      scratch_types=[
          pltpu.VMEM((NCHUNK, CHUNK), jnp.int32),
          pltpu.VMEM((NBUF, CHUNK, DIM), jnp.float32),
          pltpu.VMEM((NTBUF, 8, 8, CHUNK), jnp.float32),
          pltpu.SemaphoreType.DMA((NBUF,)),
          pltpu.SemaphoreType.DMA((NTBUF,)),
      ],
  )
  y = run(idx_t, table)
  return y.transpose(2, 4, 0, 1, 3).reshape(BATCH, HIST, DIM)


# speedup vs baseline: 1.0777x; 1.0777x over previous
import functools

import jax
import jax.numpy as jnp
from jax import lax
from jax.experimental import pallas as pl
from jax.experimental.pallas import tpu as pltpu
from jax.experimental.pallas import tpu_sc as plsc

NUM_EMB = 1000000
DIM = 64
BATCH = 4096
HIST = 50

NC = 2
NS = 16
NW = NC * NS
TOTAL = BATCH * HIST
B_PER_W = TOTAL // NW
CHUNK = 128
HCHUNK = CHUNK // 2
NCHUNK = B_PER_W // CHUNK
NBUF = 5
OUT_ROWS = TOTAL // 2


def _body(idx_hbm, table_hbm, out_hbm, idx_v, rows_v, gsem, osem):
  wid = lax.axis_index("s") * NC + lax.axis_index("c")
  base2 = wid * (B_PER_W // 2)

  pltpu.sync_copy(idx_hbm.at[wid], idx_v)

  def start_gather(j, b):
    pltpu.async_copy(table_hbm.at[idx_v.at[j]], rows_v.at[b], gsem.at[b])

  def wait_gather(j, b):
    pltpu.make_async_copy(table_hbm.at[idx_v.at[j]], rows_v.at[b],
                          gsem.at[b]).wait()

  def _write_parts(j, b):
    dst = out_hbm.at[pl.ds(base2 + j * HCHUNK, HCHUNK), pl.ds(0, DIM)]
    dst2 = out_hbm.at[pl.ds(base2 + j * HCHUNK, HCHUNK), pl.ds(DIM, DIM)]
    src = rows_v.at[b, pl.ds(0, HCHUNK)]
    src2 = rows_v.at[b, pl.ds(HCHUNK, HCHUNK)]
    return (src, dst), (src2, dst2)

  def start_write(j, b):
    for s, d in _write_parts(j, b):
      pltpu.async_copy(s, d, osem.at[b])

  def wait_write(j, b):
    for s, d in _write_parts(j, b):
      pltpu.make_async_copy(s, d, osem.at[b]).wait()

  for b in range(NBUF):
    start_gather(b, b)

  @pl.loop(0, NCHUNK, step=NBUF)
  def _outer(j0):
    for b in range(NBUF):
      j = j0 + b
      bp = (b - 1) % NBUF
      wait_gather(j, b)
      start_write(j, b)
      if b == 0:
        @pl.when(j >= 1)
        def _():
          wait_write(j - 1, bp)

        @pl.when((j >= 1) & (j + NBUF - 1 < NCHUNK))
        def _():
          start_gather(j + NBUF - 1, bp)
      else:
        wait_write(j - 1, bp)

        @pl.when(j + NBUF - 1 < NCHUNK)
        def _():
          start_gather(j + NBUF - 1, bp)

  wait_write(NCHUNK - 1, (NCHUNK - 1) % NBUF)


@jax.jit
def kernel(indices, table):
  idxr = indices.reshape(NW, NCHUNK, HCHUNK, 2).astype(jnp.int32)
  idx = jnp.concatenate([idxr[..., 0], idxr[..., 1]], axis=-1)
  run = pl.kernel(
      _body,
      out_type=jax.ShapeDtypeStruct((OUT_ROWS, 2 * DIM), jnp.float32),
      mesh=plsc.VectorSubcoreMesh(core_axis_name="c", subcore_axis_name="s"),
      compiler_params=pltpu.CompilerParams(use_tc_tiling_on_sc=False),
      scratch_types=[
          pltpu.VMEM((NCHUNK, CHUNK), jnp.int32),
          pltpu.VMEM((NBUF, CHUNK, DIM), jnp.float32),
          pltpu.SemaphoreType.DMA((NBUF,)),
          pltpu.SemaphoreType.DMA((NBUF,)),
      ],
  )
  out = run(idx, table)
  return out.reshape(BATCH, HIST, DIM)
